# Initial kernel scaffold; baseline (speedup 1.0000x reference)
import functools

import jax
import jax.numpy as jnp
from jax.experimental import pallas as pl

_N = 10000
_D = 32
_K = 16
_QB = 200


def _knn_block_kernel(q_ref, keys_ref, o_ref):
    q = q_ref[...]
    keys = keys_ref[...]
    ksq = jnp.sum(keys * keys, axis=1)
    qk = jax.lax.dot_general(
        q, keys, (((1,), (1,)), ((), ())), preferred_element_type=jnp.float32
    )
    d = ksq[None, :] - 2.0 * qk
    colid = jax.lax.broadcasted_iota(jnp.int32, d.shape, 1)
    picks = []
    for _ in range(_K):
        m = jnp.min(d, axis=1, keepdims=True)
        idx = jnp.min(jnp.where(d == m, colid, _N), axis=1)
        picks.append(idx)
        d = jnp.where(colid == idx[:, None], jnp.inf, d)
    o_ref[...] = jnp.stack(picks, axis=1).astype(jnp.float32)


@jax.jit
def kernel(barycenters):
    grid = _N // _QB
    return pl.pallas_call(
        _knn_block_kernel,
        grid=(grid,),
        in_specs=[
            pl.BlockSpec((_QB, _D), lambda i: (i, 0)),
            pl.BlockSpec((_N, _D), lambda i: (0, 0)),
        ],
        out_specs=pl.BlockSpec((_QB, _K), lambda i: (i, 0)),
        out_shape=jax.ShapeDtypeStruct((_N, _K), jnp.float32),
    )(barycenters, barycenters)


# speedup vs baseline: 2.6462x; 2.6462x over previous
import jax
import jax.numpy as jnp
from jax.experimental import pallas as pl
from jax.experimental.pallas import tpu as pltpu

_N = 10000
_NP = 10240
_D = 32
_K = 16
_QB = 400
_CW = 1024
_NCH = _NP // _CW
_PAD_VAL = 1e6


def _knn_kernel(q_ref, keysT_ref, o_ref, d_ref, mprev_ref, iprev_ref):
    j = pl.program_id(1)
    inf = jnp.float32(jnp.inf)

    @pl.when(j == 0)
    def _init():
        q = q_ref[...]
        for c in range(_NCH):
            kT_c = keysT_ref[:, c * _CW : (c + 1) * _CW]
            ksq_c = jnp.sum(kT_c * kT_c, axis=0)
            qk_c = jax.lax.dot_general(
                q, kT_c, (((1,), (0,)), ((), ())),
                preferred_element_type=jnp.float32,
                precision=jax.lax.Precision.HIGHEST,
            )
            d_ref[:, c * _CW : (c + 1) * _CW] = ksq_c[None, :] - 2.0 * qk_c
        mprev_ref[...] = jnp.full((_QB, 1), -inf, jnp.float32)
        iprev_ref[...] = jnp.full((_QB, 1), -1, jnp.int32)
        o_ref[...] = jnp.zeros((_QB, _K), jnp.float32)

    m_prev = mprev_ref[...]
    i_prev = iprev_ref[...]
    m = jnp.full((_QB, 1), inf, jnp.float32)
    idx = jnp.full((_QB, 1), _NP, jnp.int32)
    for c in range(_NCH):
        tile = d_ref[:, c * _CW : (c + 1) * _CW]
        col = jax.lax.broadcasted_iota(jnp.int32, (_QB, _CW), 1) + c * _CW
        elig = (tile > m_prev) | ((tile == m_prev) & (col > i_prev))
        tv = jnp.where(elig, tile, inf)
        mc = jnp.min(tv, axis=1, keepdims=True)
        ic = jnp.min(jnp.where(tv == mc, col, _NP), axis=1, keepdims=True)
        better = (mc < m) | ((mc == m) & (ic < idx))
        m = jnp.where(better, mc, m)
        idx = jnp.where(better, ic, idx)
    mprev_ref[...] = m
    iprev_ref[...] = idx
    kcol = jax.lax.broadcasted_iota(jnp.int32, (_QB, _K), 1)
    o_ref[...] += jnp.where(kcol == j, idx.astype(jnp.float32), 0.0)


@jax.jit
def kernel(barycenters):
    keys_pad = jnp.concatenate(
        [barycenters, jnp.full((_NP - _N, _D), _PAD_VAL, jnp.float32)], axis=0
    )
    keys_t = keys_pad.T
    grid = (_N // _QB, _K)
    return pl.pallas_call(
        _knn_kernel,
        grid=grid,
        in_specs=[
            pl.BlockSpec((_QB, _D), lambda i, j: (i, 0)),
            pl.BlockSpec((_D, _NP), lambda i, j: (0, 0)),
        ],
        out_specs=pl.BlockSpec((_QB, _K), lambda i, j: (i, 0)),
        out_shape=jax.ShapeDtypeStruct((_N, _K), jnp.float32),
        scratch_shapes=[
            pltpu.VMEM((_QB, _NP), jnp.float32),
            pltpu.VMEM((_QB, 1), jnp.float32),
            pltpu.VMEM((_QB, 1), jnp.int32),
        ],
    )(barycenters, keys_t)


# speedup vs baseline: 5.4579x; 2.0625x over previous
import functools

import jax
import jax.numpy as jnp
from jax import lax
from jax.experimental import pallas as pl
from jax.experimental.pallas import tpu as pltpu
from jax.experimental.pallas import tpu_sc as plsc

_N = 10000
_NP = 10240
_D = 32
_K = 16
_QB = 400
_CW = 1024
_NCH = _NP // _CW
_C = _NP // _K
_PAD_VAL = 1e6

_NW = 32
_ROWS_PER_W = (_N + _NW - 1) // _NW


def _tc_kernel(q_ref, keysT_ref, keysTp_ref, d_ref, cid_ref, cols_ref,
               r_ref, mprev_ref, iprev_ref):
    j = pl.program_id(1)
    inf = jnp.float32(jnp.inf)

    @pl.when(j == 0)
    def _init():
        q = q_ref[...]
        for c in range(_NCH):
            kT_c = keysT_ref[:, c * _CW : (c + 1) * _CW]
            ksq_c = jnp.sum(kT_c * kT_c, axis=0)
            qk_c = jax.lax.dot_general(
                q, kT_c, (((1,), (0,)), ((), ())),
                preferred_element_type=jnp.float32,
                precision=jax.lax.Precision.HIGHEST,
            )
            d_ref[:, c * _CW : (c + 1) * _CW] = ksq_c[None, :] - 2.0 * qk_c
        rmin = None
        for s in range(_K):
            kp_s = keysTp_ref[:, s * _C : (s + 1) * _C]
            ksqp_s = jnp.sum(kp_s * kp_s, axis=0)
            qkp_s = jax.lax.dot_general(
                q, kp_s, (((1,), (0,)), ((), ())),
                preferred_element_type=jnp.float32,
                precision=jax.lax.Precision.HIGHEST,
            )
            dp_s = ksqp_s[None, :] - 2.0 * qkp_s
            rmin = dp_s if rmin is None else jnp.minimum(rmin, dp_s)
        r_ref[...] = rmin
        mprev_ref[...] = jnp.full((_QB, 1), -inf, jnp.float32)
        iprev_ref[...] = jnp.full((_QB, 1), -1, jnp.int32)
        cid_ref[...] = jnp.zeros((_QB, _K), jnp.int32)
        cols_ref[...] = jnp.zeros((_QB, _K * _K), jnp.int32)

    m_prev = mprev_ref[...]
    i_prev = iprev_ref[...]
    r = r_ref[...]
    chid = jax.lax.broadcasted_iota(jnp.int32, (_QB, _C), 1)
    elig = (r > m_prev) | ((r == m_prev) & (chid > i_prev))
    tv = jnp.where(elig, r, inf)
    mc = jnp.min(tv, axis=1, keepdims=True)
    ic = jnp.min(jnp.where(tv == mc, chid, _C), axis=1, keepdims=True)
    mprev_ref[...] = mc
    iprev_ref[...] = ic
    kcol = jax.lax.broadcasted_iota(jnp.int32, (_QB, _K), 1)
    cid_ref[...] += jnp.where(kcol == j, ic, 0)
    col256 = jax.lax.broadcasted_iota(jnp.int32, (_QB, _K * _K), 1)
    cols_ref[...] += jnp.where(
        col256 // _K == j, ic * _K + (col256 % _K), 0
    )


def _sc_merge_kernel(d_hbm, cols_hbm, out_hbm, rowbuf, colbuf, outbuf):
    wid = lax.axis_index("s") * 2 + lax.axis_index("c")

    def body(i, carry):
        row = i * _NW + wid

        @pl.when(row < _N)
        def _():
            pltpu.sync_copy(d_hbm.at[row], rowbuf)
            pltpu.sync_copy(cols_hbm.at[row], colbuf)
            bk = None
            bv = None
            for j in range(_K):
                cols = colbuf[pl.ds(j * _K, _K)]
                vals = plsc.load_gather(rowbuf, [cols])
                k2, v2 = plsc.sort_key_val(vals, cols)
                if j == 0:
                    bk, bv = k2, v2
                else:
                    rk = lax.rev(k2, (0,))
                    rv = lax.rev(v2, (0,))
                    ck = jnp.minimum(bk, rk)
                    cv = jnp.where(bk <= rk, bv, rv)
                    bk, bv = plsc.sort_key_val(ck, cv)
            outbuf[...] = bv.astype(jnp.float32)
            pltpu.sync_copy(outbuf, out_hbm.at[row])

        return carry

    jax.lax.fori_loop(0, _ROWS_PER_W, body, 0)


@functools.cache
def _get_sc_merge():
    return pl.kernel(
        _sc_merge_kernel,
        out_type=jax.ShapeDtypeStruct((_N, _K), jnp.float32),
        mesh=plsc.VectorSubcoreMesh(core_axis_name="c", subcore_axis_name="s"),
        compiler_params=pltpu.CompilerParams(needs_layout_passes=False),
        scratch_types=[
            pltpu.VMEM((_NP,), jnp.float32),
            pltpu.VMEM((_K * _K,), jnp.int32),
            pltpu.VMEM((_K,), jnp.float32),
        ],
    )


@jax.jit
def kernel(barycenters):
    keys_pad = jnp.concatenate(
        [barycenters, jnp.full((_NP - _N, _D), _PAD_VAL, jnp.float32)], axis=0
    )
    keys_t = keys_pad.T
    keys_tp = keys_t.reshape(_D, _C, _K).transpose(0, 2, 1).reshape(_D, _NP)
    grid = (_N // _QB, _K)
    d_full, cids, cols = pl.pallas_call(
        _tc_kernel,
        grid=grid,
        in_specs=[
            pl.BlockSpec((_QB, _D), lambda i, j: (i, 0)),
            pl.BlockSpec((_D, _NP), lambda i, j: (0, 0)),
            pl.BlockSpec((_D, _NP), lambda i, j: (0, 0)),
        ],
        out_specs=[
            pl.BlockSpec((_QB, _NP), lambda i, j: (i, 0)),
            pl.BlockSpec((_QB, _K), lambda i, j: (i, 0)),
            pl.BlockSpec((_QB, _K * _K), lambda i, j: (i, 0)),
        ],
        out_shape=[
            jax.ShapeDtypeStruct((_N, _NP), jnp.float32),
            jax.ShapeDtypeStruct((_N, _K), jnp.int32),
            jax.ShapeDtypeStruct((_N, _K * _K), jnp.int32),
        ],
        scratch_shapes=[
            pltpu.VMEM((_QB, _C), jnp.float32),
            pltpu.VMEM((_QB, 1), jnp.float32),
            pltpu.VMEM((_QB, 1), jnp.int32),
        ],
    )(barycenters, keys_t, keys_tp)
    del cids
    return _get_sc_merge()(d_full, cols)


# speedup vs baseline: 9.2364x; 1.6923x over previous
import functools

import jax
import jax.numpy as jnp
from jax import lax
from jax.experimental import pallas as pl
from jax.experimental.pallas import tpu as pltpu
from jax.experimental.pallas import tpu_sc as plsc

_N = 10000
_NP = 10240
_D = 32
_K = 16
_QB = 400
_C = _NP // _K
_PAD_VAL = 1e6

_NW = 32
_PAIRS = 158


def _tc_kernel(q_ref, keysTp_ref, d_ref, cols_ref, r_ref, mprev_ref, iprev_ref):
    j = pl.program_id(1)
    inf = jnp.float32(jnp.inf)

    @pl.when(j == 0)
    def _init():
        q = q_ref[...]
        rmin = None
        for s in range(_K):
            kp_s = keysTp_ref[:, s * _C : (s + 1) * _C]
            ksqp_s = jnp.sum(kp_s * kp_s, axis=0)
            qkp_s = jax.lax.dot_general(
                q, kp_s, (((1,), (0,)), ((), ())),
                preferred_element_type=jnp.float32,
                precision=jax.lax.Precision.HIGHEST,
            )
            dp_s = ksqp_s[None, :] - 2.0 * qkp_s
            d_ref[:, s * _C : (s + 1) * _C] = dp_s
            rmin = dp_s if rmin is None else jnp.minimum(rmin, dp_s)
        r_ref[...] = rmin
        mprev_ref[...] = jnp.full((_QB, 1), -inf, jnp.float32)
        iprev_ref[...] = jnp.full((_QB, 1), -1, jnp.int32)
        cols_ref[...] = jnp.zeros((_QB, _K * _K), jnp.int32)

    m_prev = mprev_ref[...]
    i_prev = iprev_ref[...]
    r = r_ref[...]
    chid = jax.lax.broadcasted_iota(jnp.int32, (_QB, _C), 1)
    elig = (r > m_prev) | ((r == m_prev) & (chid > i_prev))
    tv = jnp.where(elig, r, inf)
    mc = jnp.min(tv, axis=1, keepdims=True)
    ic = jnp.min(jnp.where(tv == mc, chid, _C), axis=1, keepdims=True)
    mprev_ref[...] = mc
    iprev_ref[...] = ic
    col256 = jax.lax.broadcasted_iota(jnp.int32, (_QB, _K * _K), 1)
    cols_ref[...] += jnp.where(col256 // _K == j, ic * _K + (col256 % _K), 0)


def _sc_merge_kernel(d_hbm, cols_hbm, out_hbm, rowbuf0, rowbuf1,
                     colbuf0, colbuf1, outbuf, sem0, sem1):
    wid = lax.axis_index("s") * 2 + lax.axis_index("c")
    sems = (sem0, sem1)
    rowbufs = (rowbuf0, rowbuf1)
    colbufs = (colbuf0, colbuf1)
    nmax = jnp.int32(_N - 1)

    def issue(i, b):
        row = jnp.minimum(i * _NW + wid, nmax)
        pltpu.async_copy(d_hbm.at[row], rowbufs[b], sems[b])
        pltpu.async_copy(cols_hbm.at[row], colbufs[b], sems[b])

    def drain(b):
        pltpu.make_async_copy(d_hbm.at[0], rowbufs[b], sems[b]).wait()
        pltpu.make_async_copy(cols_hbm.at[0], colbufs[b], sems[b]).wait()

    def process(i, b):
        row = i * _NW + wid

        @pl.when(row < _N)
        def _():
            bk = None
            bv = None
            for j in range(_K):
                cols = colbufs[b][pl.ds(j * _K, _K)]
                idxp = (cols % _K) * _C + cols // _K
                vals = plsc.load_gather(rowbufs[b], [idxp])
                k2, v2 = plsc.sort_key_val(vals, cols)
                if j == 0:
                    bk, bv = k2, v2
                else:
                    rk = lax.rev(k2, (0,))
                    rv = lax.rev(v2, (0,))
                    ck = jnp.minimum(bk, rk)
                    cv = jnp.where(bk <= rk, bv, rv)
                    bk, bv = plsc.sort_key_val(ck, cv)
            outbuf[...] = bv.astype(jnp.float32)
            pltpu.sync_copy(outbuf, out_hbm.at[row])

    issue(0, 0)
    issue(1, 1)

    def body(g, carry):
        for sub in range(2):
            i = 2 * g + sub
            drain(sub)
            process(i, sub)
            issue(i + 2, sub)
        return carry

    jax.lax.fori_loop(0, _PAIRS, body, 0)
    drain(0)
    drain(1)


@functools.cache
def _get_sc_merge():
    return pl.kernel(
        _sc_merge_kernel,
        out_type=jax.ShapeDtypeStruct((_N, _K), jnp.float32),
        mesh=plsc.VectorSubcoreMesh(core_axis_name="c", subcore_axis_name="s"),
        compiler_params=pltpu.CompilerParams(needs_layout_passes=False),
        scratch_types=[
            pltpu.VMEM((_NP,), jnp.float32),
            pltpu.VMEM((_NP,), jnp.float32),
            pltpu.VMEM((_K * _K,), jnp.int32),
            pltpu.VMEM((_K * _K,), jnp.int32),
            pltpu.VMEM((_K,), jnp.float32),
            pltpu.SemaphoreType.DMA,
            pltpu.SemaphoreType.DMA,
        ],
    )


@jax.jit
def kernel(barycenters):
    keys_pad = jnp.concatenate(
        [barycenters, jnp.full((_NP - _N, _D), _PAD_VAL, jnp.float32)], axis=0
    )
    keys_t = keys_pad.T
    keys_tp = keys_t.reshape(_D, _C, _K).transpose(0, 2, 1).reshape(_D, _NP)
    grid = (_N // _QB, _K)
    d_perm, cols = pl.pallas_call(
        _tc_kernel,
        grid=grid,
        in_specs=[
            pl.BlockSpec((_QB, _D), lambda i, j: (i, 0)),
            pl.BlockSpec((_D, _NP), lambda i, j: (0, 0)),
        ],
        out_specs=[
            pl.BlockSpec((_QB, _NP), lambda i, j: (i, 0)),
            pl.BlockSpec((_QB, _K * _K), lambda i, j: (i, 0)),
        ],
        out_shape=[
            jax.ShapeDtypeStruct((_N, _NP), jnp.float32),
            jax.ShapeDtypeStruct((_N, _K * _K), jnp.int32),
        ],
        scratch_shapes=[
            pltpu.VMEM((_QB, _C), jnp.float32),
            pltpu.VMEM((_QB, 1), jnp.float32),
            pltpu.VMEM((_QB, 1), jnp.int32),
        ],
    )(barycenters, keys_tp)
    return _get_sc_merge()(d_perm, cols)


# speedup vs baseline: 9.9038x; 1.0723x over previous
import functools

import jax
import jax.numpy as jnp
from jax import lax
from jax.experimental import pallas as pl
from jax.experimental.pallas import tpu as pltpu
from jax.experimental.pallas import tpu_sc as plsc

_N = 10000
_NP = 10240
_D = 32
_K = 16
_QB = 400
_C = _NP // _K
_PAD_VAL = 1e6

_NW = 32
_PAIRS = 158


def _tc_dist_kernel(q_ref, keysTp_ref, d_ref, r_ref):
    q = q_ref[...]
    rmin = None
    for s in range(_K):
        kp_s = keysTp_ref[:, s * _C : (s + 1) * _C]
        ksqp_s = jnp.sum(kp_s * kp_s, axis=0)
        qkp_s = jax.lax.dot_general(
            q, kp_s, (((1,), (0,)), ((), ())),
            preferred_element_type=jnp.float32,
            precision=jax.lax.Precision.HIGHEST,
        )
        dp_s = ksqp_s[None, :] - 2.0 * qkp_s
        d_ref[:, s * _C : (s + 1) * _C] = dp_s
        rmin = dp_s if rmin is None else jnp.minimum(rmin, dp_s)
    r_ref[...] = rmin


def _tc_rounds_kernel(r_in_ref, cols_ref):
    inf = jnp.float32(jnp.inf)
    r = r_in_ref[...]
    chid = jax.lax.broadcasted_iota(jnp.int32, (_QB, _C), 1)
    iota16 = jax.lax.broadcasted_iota(jnp.int32, (_QB, _K), 1)
    m_prev = jnp.full((_QB, 1), -inf, jnp.float32)
    i_prev = jnp.full((_QB, 1), -1, jnp.int32)
    pieces = []
    for _ in range(_K):
        elig = (r > m_prev) | ((r == m_prev) & (chid > i_prev))
        tv = jnp.where(elig, r, inf)
        mc = jnp.min(tv, axis=1, keepdims=True)
        ic = jnp.min(jnp.where(tv == mc, chid, _C), axis=1, keepdims=True)
        m_prev = mc
        i_prev = ic
        pieces.append(ic * _K + iota16)
    cols_ref[...] = jnp.concatenate(pieces, axis=1)


def _sc_merge_kernel(d_hbm, cols_hbm, out_hbm, rowbuf0, rowbuf1,
                     colbuf0, colbuf1, outbuf, sem0, sem1):
    wid = lax.axis_index("s") * 2 + lax.axis_index("c")
    sems = (sem0, sem1)
    rowbufs = (rowbuf0, rowbuf1)
    colbufs = (colbuf0, colbuf1)
    nmax = jnp.int32(_N - 1)

    def issue(i, b):
        row = jnp.minimum(i * _NW + wid, nmax)
        pltpu.async_copy(d_hbm.at[row], rowbufs[b], sems[b])
        pltpu.async_copy(cols_hbm.at[row], colbufs[b], sems[b])

    def drain(b):
        pltpu.make_async_copy(d_hbm.at[0], rowbufs[b], sems[b]).wait()
        pltpu.make_async_copy(cols_hbm.at[0], colbufs[b], sems[b]).wait()

    def process(i, b):
        row = i * _NW + wid

        @pl.when(row < _N)
        def _():
            bk = None
            bv = None
            for j in range(_K):
                cols = colbufs[b][pl.ds(j * _K, _K)]
                idxp = (cols % _K) * _C + cols // _K
                vals = plsc.load_gather(rowbufs[b], [idxp])
                k2, v2 = plsc.sort_key_val(vals, cols)
                if j == 0:
                    bk, bv = k2, v2
                else:
                    rk = lax.rev(k2, (0,))
                    rv = lax.rev(v2, (0,))
                    ck = jnp.minimum(bk, rk)
                    cv = jnp.where(bk <= rk, bv, rv)
                    bk, bv = plsc.sort_key_val(ck, cv)
            outbuf[...] = bv.astype(jnp.float32)
            pltpu.sync_copy(outbuf, out_hbm.at[row])

    issue(0, 0)
    issue(1, 1)

    def body(g, carry):
        for sub in range(2):
            i = 2 * g + sub
            drain(sub)
            process(i, sub)
            issue(i + 2, sub)
        return carry

    jax.lax.fori_loop(0, _PAIRS, body, 0)
    drain(0)
    drain(1)


@functools.cache
def _get_sc_merge():
    return pl.kernel(
        _sc_merge_kernel,
        out_type=jax.ShapeDtypeStruct((_N, _K), jnp.float32),
        mesh=plsc.VectorSubcoreMesh(core_axis_name="c", subcore_axis_name="s"),
        compiler_params=pltpu.CompilerParams(needs_layout_passes=False),
        scratch_types=[
            pltpu.VMEM((_NP,), jnp.float32),
            pltpu.VMEM((_NP,), jnp.float32),
            pltpu.VMEM((_K * _K,), jnp.int32),
            pltpu.VMEM((_K * _K,), jnp.int32),
            pltpu.VMEM((_K,), jnp.float32),
            pltpu.SemaphoreType.DMA,
            pltpu.SemaphoreType.DMA,
        ],
    )


@jax.jit
def kernel(barycenters):
    keys_pad = jnp.concatenate(
        [barycenters, jnp.full((_NP - _N, _D), _PAD_VAL, jnp.float32)], axis=0
    )
    keys_t = keys_pad.T
    keys_tp = keys_t.reshape(_D, _C, _K).transpose(0, 2, 1).reshape(_D, _NP)
    grid = (_N // _QB,)
    d_perm, r_mins = pl.pallas_call(
        _tc_dist_kernel,
        grid=grid,
        in_specs=[
            pl.BlockSpec((_QB, _D), lambda i: (i, 0)),
            pl.BlockSpec((_D, _NP), lambda i: (0, 0)),
        ],
        out_specs=[
            pl.BlockSpec((_QB, _NP), lambda i: (i, 0)),
            pl.BlockSpec((_QB, _C), lambda i: (i, 0)),
        ],
        out_shape=[
            jax.ShapeDtypeStruct((_N, _NP), jnp.float32),
            jax.ShapeDtypeStruct((_N, _C), jnp.float32),
        ],
    )(barycenters, keys_tp)
    cols = pl.pallas_call(
        _tc_rounds_kernel,
        grid=grid,
        in_specs=[pl.BlockSpec((_QB, _C), lambda i: (i, 0))],
        out_specs=pl.BlockSpec((_QB, _K * _K), lambda i: (i, 0)),
        out_shape=jax.ShapeDtypeStruct((_N, _K * _K), jnp.int32),
    )(r_mins)
    return _get_sc_merge()(d_perm, cols)


# speedup vs baseline: 11.7292x; 1.1843x over previous
import functools

import jax
import jax.numpy as jnp
from jax import lax
from jax.experimental import pallas as pl
from jax.experimental.pallas import tpu as pltpu
from jax.experimental.pallas import tpu_sc as plsc

_N = 10000
_NP = 10240
_D = 32
_K = 16
_QB = 400
_C = _NP // _K
_PAD_VAL = 1e6

_NW = 32
_PAIRS = 158


def _tc_dist_kernel(q_ref, keysTp_ref, d_ref, r_ref):
    q = q_ref[...]
    rmin = None
    for s in range(_K):
        kp_s = keysTp_ref[:, s * _C : (s + 1) * _C]
        ksqp_s = jnp.sum(kp_s * kp_s, axis=0)
        qkp_s = jax.lax.dot_general(
            q, kp_s, (((1,), (0,)), ((), ())),
            preferred_element_type=jnp.float32,
            precision=jax.lax.Precision.HIGHEST,
        )
        dp_s = ksqp_s[None, :] - 2.0 * qkp_s
        d_ref[:, s * _C : (s + 1) * _C] = dp_s
        rmin = dp_s if rmin is None else jnp.minimum(rmin, dp_s)
    r_ref[...] = rmin


def _tc_rounds_kernel(r_in_ref, cols_ref):
    inf = jnp.float32(jnp.inf)
    r = r_in_ref[...]
    chid = jax.lax.broadcasted_iota(jnp.int32, (_QB, _C), 1)
    iota16 = jax.lax.broadcasted_iota(jnp.int32, (_QB, _K), 1)
    m_prev = jnp.full((_QB, 1), -inf, jnp.float32)
    i_prev = jnp.full((_QB, 1), -1, jnp.int32)
    pieces = []
    for _ in range(_K):
        elig = (r > m_prev) | ((r == m_prev) & (chid > i_prev))
        tv = jnp.where(elig, r, inf)
        mc = jnp.min(tv, axis=1, keepdims=True)
        ic = jnp.min(jnp.where(tv == mc, chid, _C), axis=1, keepdims=True)
        m_prev = mc
        i_prev = ic
        pieces.append(ic * _K + iota16)
    cols_ref[...] = jnp.concatenate(pieces, axis=1)


def _sc_merge_kernel(nseg, d_hbm, cols_hbm, out_hbm, rowbuf0, rowbuf1,
                     colbuf0, colbuf1, outbuf, sem0, sem1):
    wid = lax.axis_index("s") * 2 + lax.axis_index("c")
    sems = (sem0, sem1)
    rowbufs = (rowbuf0, rowbuf1)
    colbufs = (colbuf0, colbuf1)
    nmax = jnp.int32(nseg - 1)
    npairs = (nseg + 2 * _NW - 1) // (2 * _NW)

    def issue(i, b):
        row = jnp.minimum(i * _NW + wid, nmax)
        pltpu.async_copy(d_hbm.at[row], rowbufs[b], sems[b])
        pltpu.async_copy(cols_hbm.at[row], colbufs[b], sems[b])

    def drain(b):
        pltpu.make_async_copy(d_hbm.at[0], rowbufs[b], sems[b]).wait()
        pltpu.make_async_copy(cols_hbm.at[0], colbufs[b], sems[b]).wait()

    def process(i, b):
        row = i * _NW + wid

        @pl.when(row < nseg)
        def _():
            bk = None
            bv = None
            for j in range(_K):
                cols = colbufs[b][pl.ds(j * _K, _K)]
                idxp = (cols % _K) * _C + cols // _K
                vals = plsc.load_gather(rowbufs[b], [idxp])
                k2, v2 = plsc.sort_key_val(vals, cols)
                if j == 0:
                    bk, bv = k2, v2
                else:
                    rk = lax.rev(k2, (0,))
                    rv = lax.rev(v2, (0,))
                    ck = jnp.minimum(bk, rk)
                    cv = jnp.where(bk <= rk, bv, rv)
                    bk, bv = plsc.sort_key_val(ck, cv)
            outbuf[...] = bv.astype(jnp.float32)
            pltpu.sync_copy(outbuf, out_hbm.at[row])

    issue(0, 0)
    issue(1, 1)

    def body(g, carry):
        for sub in range(2):
            i = 2 * g + sub
            drain(sub)
            process(i, sub)
            issue(i + 2, sub)
        return carry

    jax.lax.fori_loop(0, npairs, body, 0)
    drain(0)
    drain(1)


@functools.cache
def _get_sc_merge(nseg):
    return pl.kernel(
        functools.partial(_sc_merge_kernel, nseg),
        out_type=jax.ShapeDtypeStruct((nseg, _K), jnp.float32),
        mesh=plsc.VectorSubcoreMesh(core_axis_name="c", subcore_axis_name="s"),
        compiler_params=pltpu.CompilerParams(needs_layout_passes=False),
        scratch_types=[
            pltpu.VMEM((_NP,), jnp.float32),
            pltpu.VMEM((_NP,), jnp.float32),
            pltpu.VMEM((_K * _K,), jnp.int32),
            pltpu.VMEM((_K * _K,), jnp.int32),
            pltpu.VMEM((_K,), jnp.float32),
            pltpu.SemaphoreType.DMA,
            pltpu.SemaphoreType.DMA,
        ],
    )


@jax.jit
def kernel(barycenters):
    keys_pad = jnp.concatenate(
        [barycenters, jnp.full((_NP - _N, _D), _PAD_VAL, jnp.float32)], axis=0
    )
    keys_t = keys_pad.T
    keys_tp = keys_t.reshape(_D, _C, _K).transpose(0, 2, 1).reshape(_D, _NP)
    nseg = 2000
    grid = (nseg // _QB,)
    outs = []
    for g in range(_N // nseg):
        q_g = jax.lax.slice_in_dim(barycenters, g * nseg, (g + 1) * nseg)
        d_perm, r_mins = pl.pallas_call(
            _tc_dist_kernel,
            grid=grid,
            in_specs=[
                pl.BlockSpec((_QB, _D), lambda i: (i, 0)),
                pl.BlockSpec((_D, _NP), lambda i: (0, 0)),
            ],
            out_specs=[
                pl.BlockSpec((_QB, _NP), lambda i: (i, 0)),
                pl.BlockSpec((_QB, _C), lambda i: (i, 0)),
            ],
            out_shape=[
                jax.ShapeDtypeStruct((nseg, _NP), jnp.float32),
                jax.ShapeDtypeStruct((nseg, _C), jnp.float32),
            ],
        )(q_g, keys_tp)
        cols = pl.pallas_call(
            _tc_rounds_kernel,
            grid=grid,
            in_specs=[pl.BlockSpec((_QB, _C), lambda i: (i, 0))],
            out_specs=pl.BlockSpec((_QB, _K * _K), lambda i: (i, 0)),
            out_shape=jax.ShapeDtypeStruct((nseg, _K * _K), jnp.int32),
        )(r_mins)
        outs.append(_get_sc_merge(nseg)(d_perm, cols))
    return jnp.concatenate(outs, axis=0)
